# Initial kernel scaffold; baseline (speedup 1.0000x reference)
import jax
import jax.numpy as jnp
from jax.experimental import pallas as pl


def kernel(V, E, X, W1, b1, W2, b2):
    raise NotImplementedError("write your pallas kernel here")


# speedup vs baseline: 10.6920x; 10.6920x over previous
import functools

import jax
import jax.numpy as jnp
from jax import lax
from jax.experimental import pallas as pl
from jax.experimental.pallas import tpu as pltpu
from jax.experimental.pallas import tpu_sc as plsc

N = 10000
D = 128
NE = 320000

NC = 2
NS = 16
NW = NC * NS

EPT = NE // NW
CH = 80
NIT = EPT // CH
RPT = 1000
NZW = N // RPT
ZR = 200
ZBUF1 = 1008

_MESH = plsc.VectorSubcoreMesh(
    core_axis_name="c", subcore_axis_name="s", num_cores=NC, num_subcores=NS
)

_f32 = jnp.float32


def _sc_deg_body(dst_hbm, out_hbm, idx_v, ones_v, zbuf, acc):
    c = lax.axis_index("c")
    s = lax.axis_index("s")

    def fill_ones(i, _):
        ones_v[pl.ds(i * 16, 16)] = jnp.ones((16,), _f32)
        return 0

    lax.fori_loop(0, CH // 16, fill_ones, 0)

    def fill_zero(i, _):
        zbuf[pl.ds(i * 16, 16)] = jnp.zeros((16,), _f32)
        return 0

    lax.fori_loop(0, ZBUF1 // 16, fill_zero, 0)

    @pl.when(s < NZW)
    def _zero():
        pltpu.sync_copy(zbuf.at[pl.ds(0, RPT)], acc.at[pl.ds(s * RPT, RPT)])

    plsc.subcore_barrier()

    ebase = (c * NS + s) * EPT

    def body(it, _):
        off = ebase + it * CH
        pltpu.sync_copy(dst_hbm.at[pl.ds(off, CH)], idx_v)
        pltpu.sync_copy(ones_v, acc.at[idx_v], add=True)
        return 0

    lax.fori_loop(0, NIT, body, 0)
    plsc.subcore_barrier()

    @pl.when(s < NZW)
    def _writeout():
        pltpu.sync_copy(acc.at[pl.ds(s * RPT, RPT)], zbuf.at[pl.ds(0, RPT)])
        pltpu.sync_copy(zbuf.at[pl.ds(0, RPT)], out_hbm.at[pl.ds(c * N + s * RPT, RPT)])


def _sc_segsum_body(g_hbm, src_hbm, dst_hbm, out_hbm, src_v, dst_v, rows_v, zbuf, acc, sem):
    c = lax.axis_index("c")
    s = lax.axis_index("s")

    def fill_zero(i, _):
        def inner(j, _):
            zbuf[i, pl.ds(j * 16, 16)] = jnp.zeros((16,), _f32)
            return 0

        lax.fori_loop(0, D // 16, inner, 0)
        return 0

    lax.fori_loop(0, ZR, fill_zero, 0)

    @pl.when(s < NZW)
    def _zero():
        for r in range(RPT // ZR):
            pltpu.sync_copy(zbuf, acc.at[pl.ds(s * RPT + r * ZR, ZR), :])

    plsc.subcore_barrier()

    ebase = (c * NS + s) * EPT

    def body(it, _):
        off = ebase + it * CH
        pltpu.sync_copy(src_hbm.at[pl.ds(off, CH)], src_v)
        pltpu.sync_copy(dst_hbm.at[pl.ds(off, CH)], dst_v)
        pltpu.async_copy(g_hbm.at[src_v], rows_v, sem).wait()
        pltpu.sync_copy(rows_v, acc.at[dst_v], add=True)
        return 0

    lax.fori_loop(0, NIT, body, 0)
    plsc.subcore_barrier()

    @pl.when(s < NZW)
    def _writeout():
        pltpu.sync_copy(
            acc.at[pl.ds(s * RPT, RPT), :], out_hbm.at[c, pl.ds(s * RPT, RPT), :]
        )


def _make_sc_deg(interpret=False):
    return pl.kernel(
        _sc_deg_body,
        out_type=jax.ShapeDtypeStruct((NC * N,), _f32),
        mesh=_MESH,
        scratch_types=[
            pltpu.VMEM((CH,), jnp.int32),
            pltpu.VMEM((CH,), _f32),
            pltpu.VMEM((ZBUF1,), _f32),
            pltpu.VMEM_SHARED((N,), _f32),
        ],
        interpret=interpret,
    )


def _make_sc_segsum(interpret=False):
    return pl.kernel(
        _sc_segsum_body,
        out_type=jax.ShapeDtypeStruct((NC, N, D), _f32),
        mesh=_MESH,
        scratch_types=[
            pltpu.VMEM((CH,), jnp.int32),
            pltpu.VMEM((CH,), jnp.int32),
            pltpu.VMEM((CH, D), _f32),
            pltpu.VMEM((ZR, D), _f32),
            pltpu.VMEM_SHARED((N, D), _f32),
            pltpu.SemaphoreType.DMA,
        ],
        interpret=interpret,
    )


_sc_deg = _make_sc_deg()
_sc_segsum = _make_sc_segsum()


_TCB = 1000
_GRID = N // _TCB


def _tc1_body(x_ref, w_ref, d0_ref, d1_ref, z_ref, n_ref):
    deg = jnp.maximum(d0_ref[...] + d1_ref[...], 1.0)
    norm = lax.rsqrt(deg)
    z = jnp.dot(x_ref[...], w_ref[...], preferred_element_type=_f32)
    z_ref[...] = z * norm
    n_ref[...] = norm


def _tc1(x, w1, d0, d1):
    return pl.pallas_call(
        _tc1_body,
        grid=(_GRID,),
        in_specs=[
            pl.BlockSpec((_TCB, D), lambda i: (i, 0)),
            pl.BlockSpec((D, D), lambda i: (0, 0)),
            pl.BlockSpec((_TCB, 1), lambda i: (i, 0)),
            pl.BlockSpec((_TCB, 1), lambda i: (i, 0)),
        ],
        out_specs=[
            pl.BlockSpec((_TCB, D), lambda i: (i, 0)),
            pl.BlockSpec((_TCB, 1), lambda i: (i, 0)),
        ],
        out_shape=[
            jax.ShapeDtypeStruct((N, D), _f32),
            jax.ShapeDtypeStruct((N, 1), _f32),
        ],
    )(x, w1, d0, d1)


def _tc2_body(p0_ref, p1_ref, n_ref, b_ref, w_ref, z_ref):
    norm = n_ref[...]
    h = jnp.maximum((p0_ref[...] + p1_ref[...]) * norm + b_ref[...], 0.0)
    z_ref[...] = jnp.dot(h, w_ref[...], preferred_element_type=_f32) * norm


def _tc2(p0, p1, norm, b1, w2):
    return pl.pallas_call(
        _tc2_body,
        grid=(_GRID,),
        in_specs=[
            pl.BlockSpec((_TCB, D), lambda i: (i, 0)),
            pl.BlockSpec((_TCB, D), lambda i: (i, 0)),
            pl.BlockSpec((_TCB, 1), lambda i: (i, 0)),
            pl.BlockSpec((1, D), lambda i: (0, 0)),
            pl.BlockSpec((D, D), lambda i: (0, 0)),
        ],
        out_specs=pl.BlockSpec((_TCB, D), lambda i: (i, 0)),
        out_shape=jax.ShapeDtypeStruct((N, D), _f32),
    )(p0, p1, norm, b1, w2)


def _tc3_body(p0_ref, p1_ref, n_ref, b_ref, o_ref):
    o_ref[...] = (p0_ref[...] + p1_ref[...]) * n_ref[...] + b_ref[...]


def _tc3(p0, p1, norm, b2):
    return pl.pallas_call(
        _tc3_body,
        grid=(_GRID,),
        in_specs=[
            pl.BlockSpec((_TCB, D), lambda i: (i, 0)),
            pl.BlockSpec((_TCB, D), lambda i: (i, 0)),
            pl.BlockSpec((_TCB, 1), lambda i: (i, 0)),
            pl.BlockSpec((1, D), lambda i: (0, 0)),
        ],
        out_specs=pl.BlockSpec((_TCB, D), lambda i: (i, 0)),
        out_shape=jax.ShapeDtypeStruct((N, D), _f32),
    )(p0, p1, norm, b2)


def kernel(V, E, X, W1, b1, W2, b2):
    src = E[0]
    dst = E[1]
    degp = _sc_deg(dst).reshape(NC, N, 1)
    d0 = degp[0]
    d1 = degp[1]
    z1, norm = _tc1(X, W1, d0, d1)
    p1 = _sc_segsum(z1, src, dst)
    z2 = _tc2(p1[0], p1[1], norm, b1.reshape(1, D), W2)
    p2 = _sc_segsum(z2, src, dst)
    return _tc3(p2[0], p2[1], norm, b2.reshape(1, D))


# speedup vs baseline: 17.6230x; 1.6482x over previous
import functools

import jax
import jax.numpy as jnp
from jax import lax
from jax.experimental import pallas as pl
from jax.experimental.pallas import tpu as pltpu
from jax.experimental.pallas import tpu_sc as plsc

N = 10000
D = 128
NE = 320000

NC = 2
NS = 16
NW = NC * NS

EPT = NE // NW
CH = 80
NIT = EPT // CH
RPT = 1000
NZW = N // RPT
ZR = 40
ZBUF1 = 1008

_MESH = plsc.VectorSubcoreMesh(
    core_axis_name="c", subcore_axis_name="s", num_cores=NC, num_subcores=NS
)

_f32 = jnp.float32


def _sc_deg_body(dst_hbm, out_hbm, idx_v, ones_v, zbuf, acc):
    c = lax.axis_index("c")
    s = lax.axis_index("s")

    def fill_ones(i, _):
        ones_v[pl.ds(i * 16, 16)] = jnp.ones((16,), _f32)
        return 0

    lax.fori_loop(0, CH // 16, fill_ones, 0)

    def fill_zero(i, _):
        zbuf[pl.ds(i * 16, 16)] = jnp.zeros((16,), _f32)
        return 0

    lax.fori_loop(0, ZBUF1 // 16, fill_zero, 0)

    @pl.when(s < NZW)
    def _zero():
        pltpu.sync_copy(zbuf.at[pl.ds(0, RPT)], acc.at[pl.ds(s * RPT, RPT)])

    plsc.subcore_barrier()

    ebase = (c * NS + s) * EPT

    def body(it, _):
        off = ebase + it * CH
        pltpu.sync_copy(dst_hbm.at[pl.ds(off, CH)], idx_v)
        pltpu.sync_copy(ones_v, acc.at[idx_v], add=True)
        return 0

    lax.fori_loop(0, NIT, body, 0)
    plsc.subcore_barrier()

    @pl.when(s < NZW)
    def _writeout():
        pltpu.sync_copy(acc.at[pl.ds(s * RPT, RPT)], zbuf.at[pl.ds(0, RPT)])
        pltpu.sync_copy(zbuf.at[pl.ds(0, RPT)], out_hbm.at[pl.ds(c * N + s * RPT, RPT)])


def _sc_segsum_body(g_hbm, src_hbm, dst_hbm, out_hbm, src_v, dst_v, rows_a, rows_b, zbuf, acc, sem_a, sem_b):
    c = lax.axis_index("c")
    s = lax.axis_index("s")

    ebase = (c * NS + s) * EPT
    pltpu.sync_copy(src_hbm.at[pl.ds(ebase, EPT)], src_v)
    pltpu.sync_copy(dst_hbm.at[pl.ds(ebase, EPT)], dst_v)

    def fill_zero(i, _):
        def inner(j, _):
            zbuf[i, pl.ds(j * 16, 16)] = jnp.zeros((16,), _f32)
            return 0

        lax.fori_loop(0, D // 16, inner, 0)
        return 0

    lax.fori_loop(0, ZR, fill_zero, 0)

    @pl.when(s < NZW)
    def _zero():
        for r in range(RPT // ZR):
            pltpu.sync_copy(zbuf, acc.at[pl.ds(s * RPT + r * ZR, ZR), :])

    plsc.subcore_barrier()

    def gather(j, buf, sem):
        return pltpu.async_copy(g_hbm.at[src_v.at[pl.ds(j * CH, CH)]], buf, sem)

    def scatter(j, buf):
        pltpu.sync_copy(buf, acc.at[dst_v.at[pl.ds(j * CH, CH)]], add=True)

    gather(0, rows_a, sem_a)

    def body(g, _):
        j = 2 * g
        pltpu.make_async_copy(g_hbm.at[src_v.at[pl.ds(0, CH)]], rows_a, sem_a).wait()
        db = gather(j + 1, rows_b, sem_b)
        scatter(j, rows_a)
        db.wait()
        gather(j + 2, rows_a, sem_a)
        scatter(j + 1, rows_b)
        return 0

    lax.fori_loop(0, (NIT - 1) // 2, body, 0)
    pltpu.make_async_copy(g_hbm.at[src_v.at[pl.ds(0, CH)]], rows_a, sem_a).wait()
    scatter(NIT - 1, rows_a)
    plsc.subcore_barrier()

    @pl.when(s < NZW)
    def _writeout():
        pltpu.sync_copy(
            acc.at[pl.ds(s * RPT, RPT), :], out_hbm.at[c, pl.ds(s * RPT, RPT), :]
        )


def _make_sc_deg(interpret=False):
    return pl.kernel(
        _sc_deg_body,
        out_type=jax.ShapeDtypeStruct((NC * N,), _f32),
        mesh=_MESH,
        scratch_types=[
            pltpu.VMEM((CH,), jnp.int32),
            pltpu.VMEM((CH,), _f32),
            pltpu.VMEM((ZBUF1,), _f32),
            pltpu.VMEM_SHARED((N,), _f32),
        ],
        interpret=interpret,
    )


def _make_sc_segsum(interpret=False):
    return pl.kernel(
        _sc_segsum_body,
        out_type=jax.ShapeDtypeStruct((NC, N, D), _f32),
        mesh=_MESH,
        scratch_types=[
            pltpu.VMEM((EPT,), jnp.int32),
            pltpu.VMEM((EPT,), jnp.int32),
            pltpu.VMEM((CH, D), _f32),
            pltpu.VMEM((CH, D), _f32),
            pltpu.VMEM((ZR, D), _f32),
            pltpu.VMEM_SHARED((N, D), _f32),
            pltpu.SemaphoreType.DMA,
            pltpu.SemaphoreType.DMA,
        ],
        interpret=interpret,
    )


_sc_deg = _make_sc_deg()
_sc_segsum = _make_sc_segsum()


_TCB = 1000
_GRID = N // _TCB


def _tc1_body(x_ref, w_ref, d0_ref, d1_ref, z_ref, n_ref):
    deg = jnp.maximum(d0_ref[...] + d1_ref[...], 1.0)
    norm = lax.rsqrt(deg)
    z = jnp.dot(x_ref[...], w_ref[...], preferred_element_type=_f32)
    z_ref[...] = z * norm
    n_ref[...] = norm


def _tc1(x, w1, d0, d1):
    return pl.pallas_call(
        _tc1_body,
        grid=(_GRID,),
        in_specs=[
            pl.BlockSpec((_TCB, D), lambda i: (i, 0)),
            pl.BlockSpec((D, D), lambda i: (0, 0)),
            pl.BlockSpec((_TCB, 1), lambda i: (i, 0)),
            pl.BlockSpec((_TCB, 1), lambda i: (i, 0)),
        ],
        out_specs=[
            pl.BlockSpec((_TCB, D), lambda i: (i, 0)),
            pl.BlockSpec((_TCB, 1), lambda i: (i, 0)),
        ],
        out_shape=[
            jax.ShapeDtypeStruct((N, D), _f32),
            jax.ShapeDtypeStruct((N, 1), _f32),
        ],
    )(x, w1, d0, d1)


def _tc2_body(p0_ref, p1_ref, n_ref, b_ref, w_ref, z_ref):
    norm = n_ref[...]
    h = jnp.maximum((p0_ref[...] + p1_ref[...]) * norm + b_ref[...], 0.0)
    z_ref[...] = jnp.dot(h, w_ref[...], preferred_element_type=_f32) * norm


def _tc2(p0, p1, norm, b1, w2):
    return pl.pallas_call(
        _tc2_body,
        grid=(_GRID,),
        in_specs=[
            pl.BlockSpec((_TCB, D), lambda i: (i, 0)),
            pl.BlockSpec((_TCB, D), lambda i: (i, 0)),
            pl.BlockSpec((_TCB, 1), lambda i: (i, 0)),
            pl.BlockSpec((1, D), lambda i: (0, 0)),
            pl.BlockSpec((D, D), lambda i: (0, 0)),
        ],
        out_specs=pl.BlockSpec((_TCB, D), lambda i: (i, 0)),
        out_shape=jax.ShapeDtypeStruct((N, D), _f32),
    )(p0, p1, norm, b1, w2)


def _tc3_body(p0_ref, p1_ref, n_ref, b_ref, o_ref):
    o_ref[...] = (p0_ref[...] + p1_ref[...]) * n_ref[...] + b_ref[...]


def _tc3(p0, p1, norm, b2):
    return pl.pallas_call(
        _tc3_body,
        grid=(_GRID,),
        in_specs=[
            pl.BlockSpec((_TCB, D), lambda i: (i, 0)),
            pl.BlockSpec((_TCB, D), lambda i: (i, 0)),
            pl.BlockSpec((_TCB, 1), lambda i: (i, 0)),
            pl.BlockSpec((1, D), lambda i: (0, 0)),
        ],
        out_specs=pl.BlockSpec((_TCB, D), lambda i: (i, 0)),
        out_shape=jax.ShapeDtypeStruct((N, D), _f32),
    )(p0, p1, norm, b2)


def kernel(V, E, X, W1, b1, W2, b2):
    src = E[0]
    dst = E[1]
    degp = _sc_deg(dst).reshape(NC, N, 1)
    d0 = degp[0]
    d1 = degp[1]
    z1, norm = _tc1(X, W1, d0, d1)
    p1 = _sc_segsum(z1, src, dst)
    z2 = _tc2(p1[0], p1[1], norm, b1.reshape(1, D), W2)
    p2 = _sc_segsum(z2, src, dst)
    return _tc3(p2[0], p2[1], norm, b2.reshape(1, D))


# speedup vs baseline: 17.7456x; 1.0070x over previous
import functools

import jax
import jax.numpy as jnp
from jax import lax
from jax.experimental import pallas as pl
from jax.experimental.pallas import tpu as pltpu
from jax.experimental.pallas import tpu_sc as plsc

N = 10000
D = 128
NE = 320000

NC = 2
NS = 16
NW = NC * NS

EPT = NE // NW
CH = 80
NIT = EPT // CH
RPT = 1000
NZW = N // RPT
ZR = 40
ZBUF1 = 1008

_MESH = plsc.VectorSubcoreMesh(
    core_axis_name="c", subcore_axis_name="s", num_cores=NC, num_subcores=NS
)

_f32 = jnp.float32


def _sc_deg_body(dst_hbm, out_hbm, idx_v, ones_v, zbuf, acc):
    c = lax.axis_index("c")
    s = lax.axis_index("s")

    def fill_ones(i, _):
        ones_v[pl.ds(i * 16, 16)] = jnp.ones((16,), _f32)
        return 0

    lax.fori_loop(0, CH // 16, fill_ones, 0)

    def fill_zero(i, _):
        zbuf[pl.ds(i * 16, 16)] = jnp.zeros((16,), _f32)
        return 0

    lax.fori_loop(0, ZBUF1 // 16, fill_zero, 0)

    @pl.when(s < NZW)
    def _zero():
        pltpu.sync_copy(zbuf.at[pl.ds(0, RPT)], acc.at[pl.ds(s * RPT, RPT)])

    plsc.subcore_barrier()

    ebase = (c * NS + s) * EPT

    def body(it, _):
        off = ebase + it * CH
        pltpu.sync_copy(dst_hbm.at[pl.ds(off, CH)], idx_v)
        pltpu.sync_copy(ones_v, acc.at[idx_v], add=True)
        return 0

    lax.fori_loop(0, NIT, body, 0)
    plsc.subcore_barrier()

    @pl.when(s < NZW)
    def _writeout():
        pltpu.sync_copy(acc.at[pl.ds(s * RPT, RPT)], zbuf.at[pl.ds(0, RPT)])
        pltpu.sync_copy(zbuf.at[pl.ds(0, RPT)], out_hbm.at[pl.ds(c * N + s * RPT, RPT)])


def _sc_segsum_body(
    g_hbm, src_hbm, dst_hbm, out_hbm,
    src_v, dst_v, rows_a, rows_b, zbuf, acc, sem_a, sem_b, sem_sa, sem_sb,
):
    c = lax.axis_index("c")
    s = lax.axis_index("s")

    ebase = (c * NS + s) * EPT
    pltpu.sync_copy(src_hbm.at[pl.ds(ebase, EPT)], src_v)
    pltpu.sync_copy(dst_hbm.at[pl.ds(ebase, EPT)], dst_v)

    def fill_zero(i, _):
        def inner(j, _):
            zbuf[i, pl.ds(j * 16, 16)] = jnp.zeros((16,), _f32)
            return 0

        lax.fori_loop(0, D // 16, inner, 0)
        return 0

    lax.fori_loop(0, ZR, fill_zero, 0)

    @pl.when(s < NZW)
    def _zero():
        for r in range(RPT // ZR):
            pltpu.sync_copy(zbuf, acc.at[pl.ds(s * RPT + r * ZR, ZR), :])

    plsc.subcore_barrier()

    def gather(j, buf, sem):
        return pltpu.async_copy(g_hbm.at[src_v.at[pl.ds(j * CH, CH)]], buf, sem)

    def gather_wait(buf, sem):
        pltpu.make_async_copy(g_hbm.at[src_v.at[pl.ds(0, CH)]], buf, sem).wait()

    def scatter(j, buf, sem):
        return pltpu.async_copy(buf, acc.at[dst_v.at[pl.ds(j * CH, CH)]], sem, add=True)

    gather(0, rows_a, sem_a)
    gather(1, rows_b, sem_b)

    def body(g, _):
        j = 2 * g
        gather_wait(rows_a, sem_a)
        dsa = scatter(j, rows_a, sem_sa)
        gather_wait(rows_b, sem_b)
        dsb = scatter(j + 1, rows_b, sem_sb)
        dsa.wait()
        gather(j + 2, rows_a, sem_a)
        dsb.wait()
        gather(j + 3, rows_b, sem_b)
        return 0

    lax.fori_loop(0, (NIT - 1) // 2 - 1, body, 0)
    j_last = NIT - 3
    gather_wait(rows_a, sem_a)
    dsa = scatter(j_last, rows_a, sem_sa)
    gather_wait(rows_b, sem_b)
    dsb = scatter(j_last + 1, rows_b, sem_sb)
    dsa.wait()
    gather(j_last + 2, rows_a, sem_a)
    dsb.wait()
    gather_wait(rows_a, sem_a)
    scatter(NIT - 1, rows_a, sem_sa).wait()
    plsc.subcore_barrier()

    @pl.when(s < NZW)
    def _writeout():
        pltpu.sync_copy(
            acc.at[pl.ds(s * RPT, RPT), :], out_hbm.at[c, pl.ds(s * RPT, RPT), :]
        )


def _make_sc_deg(interpret=False):
    return pl.kernel(
        _sc_deg_body,
        out_type=jax.ShapeDtypeStruct((NC * N,), _f32),
        mesh=_MESH,
        scratch_types=[
            pltpu.VMEM((CH,), jnp.int32),
            pltpu.VMEM((CH,), _f32),
            pltpu.VMEM((ZBUF1,), _f32),
            pltpu.VMEM_SHARED((N,), _f32),
        ],
        interpret=interpret,
    )


def _make_sc_segsum(interpret=False):
    return pl.kernel(
        _sc_segsum_body,
        out_type=jax.ShapeDtypeStruct((NC, N, D), _f32),
        mesh=_MESH,
        scratch_types=[
            pltpu.VMEM((EPT,), jnp.int32),
            pltpu.VMEM((EPT,), jnp.int32),
            pltpu.VMEM((CH, D), _f32),
            pltpu.VMEM((CH, D), _f32),
            pltpu.VMEM((ZR, D), _f32),
            pltpu.VMEM_SHARED((N, D), _f32),
            pltpu.SemaphoreType.DMA,
            pltpu.SemaphoreType.DMA,
            pltpu.SemaphoreType.DMA,
            pltpu.SemaphoreType.DMA,
        ],
        interpret=interpret,
    )


_sc_deg = _make_sc_deg()
_sc_segsum = _make_sc_segsum()


_TCB = 1000
_GRID = N // _TCB


def _tc_mm_body(x_ref, w_ref, z_ref):
    z_ref[...] = jnp.dot(x_ref[...], w_ref[...], preferred_element_type=_f32)


def _tc_mm(x, w):
    return pl.pallas_call(
        _tc_mm_body,
        grid=(_GRID,),
        in_specs=[
            pl.BlockSpec((_TCB, D), lambda i: (i, 0)),
            pl.BlockSpec((D, D), lambda i: (0, 0)),
        ],
        out_specs=pl.BlockSpec((_TCB, D), lambda i: (i, 0)),
        out_shape=jax.ShapeDtypeStruct((N, D), _f32),
    )(x, w)


def _tc1_body(g_ref, d0_ref, d1_ref, z_ref, n_ref):
    deg = jnp.maximum(d0_ref[...] + d1_ref[...], 1.0)
    norm = lax.rsqrt(deg)
    z_ref[...] = g_ref[...] * norm
    n_ref[...] = norm


def _tc1(g, d0, d1):
    return pl.pallas_call(
        _tc1_body,
        grid=(_GRID,),
        in_specs=[
            pl.BlockSpec((_TCB, D), lambda i: (i, 0)),
            pl.BlockSpec((_TCB, 1), lambda i: (i, 0)),
            pl.BlockSpec((_TCB, 1), lambda i: (i, 0)),
        ],
        out_specs=[
            pl.BlockSpec((_TCB, D), lambda i: (i, 0)),
            pl.BlockSpec((_TCB, 1), lambda i: (i, 0)),
        ],
        out_shape=[
            jax.ShapeDtypeStruct((N, D), _f32),
            jax.ShapeDtypeStruct((N, 1), _f32),
        ],
    )(g, d0, d1)


def _tc2_body(p0_ref, p1_ref, n_ref, b_ref, w_ref, z_ref):
    norm = n_ref[...]
    h = jnp.maximum((p0_ref[...] + p1_ref[...]) * norm + b_ref[...], 0.0)
    z_ref[...] = jnp.dot(h, w_ref[...], preferred_element_type=_f32) * norm


def _tc2(p0, p1, norm, b1, w2):
    return pl.pallas_call(
        _tc2_body,
        grid=(_GRID,),
        in_specs=[
            pl.BlockSpec((_TCB, D), lambda i: (i, 0)),
            pl.BlockSpec((_TCB, D), lambda i: (i, 0)),
            pl.BlockSpec((_TCB, 1), lambda i: (i, 0)),
            pl.BlockSpec((1, D), lambda i: (0, 0)),
            pl.BlockSpec((D, D), lambda i: (0, 0)),
        ],
        out_specs=pl.BlockSpec((_TCB, D), lambda i: (i, 0)),
        out_shape=jax.ShapeDtypeStruct((N, D), _f32),
    )(p0, p1, norm, b1, w2)


def _tc3_body(p0_ref, p1_ref, n_ref, b_ref, o_ref):
    o_ref[...] = (p0_ref[...] + p1_ref[...]) * n_ref[...] + b_ref[...]


def _tc3(p0, p1, norm, b2):
    return pl.pallas_call(
        _tc3_body,
        grid=(_GRID,),
        in_specs=[
            pl.BlockSpec((_TCB, D), lambda i: (i, 0)),
            pl.BlockSpec((_TCB, D), lambda i: (i, 0)),
            pl.BlockSpec((_TCB, 1), lambda i: (i, 0)),
            pl.BlockSpec((1, D), lambda i: (0, 0)),
        ],
        out_specs=pl.BlockSpec((_TCB, D), lambda i: (i, 0)),
        out_shape=jax.ShapeDtypeStruct((N, D), _f32),
    )(p0, p1, norm, b2)


def kernel(V, E, X, W1, b1, W2, b2):
    src = E[0]
    dst = E[1]
    g1 = _tc_mm(X, W1)
    degp = _sc_deg(dst).reshape(NC, N, 1)
    z1, norm = _tc1(g1, degp[0], degp[1])
    p1 = _sc_segsum(z1, src, dst)
    z2 = _tc2(p1[0], p1[1], norm, b1.reshape(1, D), W2)
    p2 = _sc_segsum(z2, src, dst)
    return _tc3(p2[0], p2[1], norm, b2.reshape(1, D))


# speedup vs baseline: 20.1500x; 1.1355x over previous
import functools

import jax
import jax.numpy as jnp
from jax import lax
from jax.experimental import pallas as pl
from jax.experimental.pallas import tpu as pltpu
from jax.experimental.pallas import tpu_sc as plsc

N = 10000
D = 128
NE = 320000

NC = 2
NS = 16
NW = NC * NS

EPT = NE // NW
CH = 80
NIT = EPT // CH
RPT = 1000
NZW = N // RPT
ZR = 40
ZBUF1 = 1008

_MESH = plsc.VectorSubcoreMesh(
    core_axis_name="c", subcore_axis_name="s", num_cores=NC, num_subcores=NS
)

_f32 = jnp.float32


def _sc_deg_body(dst_hbm, out_hbm, idx_v, ones_v, zbuf, acc, sem):
    c = lax.axis_index("c")
    s = lax.axis_index("s")

    def fill_ones(i, _):
        ones_v[pl.ds(i * 16, 16)] = jnp.ones((16,), _f32)
        return 0

    lax.fori_loop(0, CH // 16, fill_ones, 0)

    def fill_zero(i, _):
        zbuf[pl.ds(i * 16, 16)] = jnp.zeros((16,), _f32)
        return 0

    lax.fori_loop(0, ZBUF1 // 16, fill_zero, 0)
    pltpu.sync_copy(dst_hbm.at[pl.ds((c * NS + s) * EPT, EPT)], idx_v)

    @pl.when(s < NZW)
    def _zero():
        pltpu.sync_copy(zbuf.at[pl.ds(0, RPT)], acc.at[pl.ds(s * RPT, RPT)])

    plsc.subcore_barrier()

    def fire(j, _):
        pltpu.async_copy(ones_v, acc.at[idx_v.at[pl.ds(j * CH, CH)]], sem, add=True)
        return 0

    lax.fori_loop(0, NIT, fire, 0)

    def drain(j, _):
        pltpu.make_async_copy(ones_v, acc.at[idx_v.at[pl.ds(0, CH)]], sem).wait()
        return 0

    lax.fori_loop(0, NIT, drain, 0)
    plsc.subcore_barrier()

    @pl.when(s < NZW)
    def _writeout():
        pltpu.sync_copy(acc.at[pl.ds(s * RPT, RPT)], zbuf.at[pl.ds(0, RPT)])
        pltpu.sync_copy(zbuf.at[pl.ds(0, RPT)], out_hbm.at[pl.ds(c * N + s * RPT, RPT)])


def _sc_segsum_body(
    g_hbm, src_hbm, dst_hbm, out_hbm,
    src_v, dst_v, rows_a, rows_b, zbuf, acc, sem_a, sem_b, sem_sa, sem_sb,
):
    c = lax.axis_index("c")
    s = lax.axis_index("s")

    ebase = (c * NS + s) * EPT
    pltpu.sync_copy(src_hbm.at[pl.ds(ebase, EPT)], src_v)
    pltpu.sync_copy(dst_hbm.at[pl.ds(ebase, EPT)], dst_v)

    def fill_zero(i, _):
        def inner(j, _):
            zbuf[i, pl.ds(j * 16, 16)] = jnp.zeros((16,), _f32)
            return 0

        lax.fori_loop(0, D // 16, inner, 0)
        return 0

    lax.fori_loop(0, ZR, fill_zero, 0)

    @pl.when(s < NZW)
    def _zero():
        for r in range(RPT // ZR):
            pltpu.sync_copy(zbuf, acc.at[pl.ds(s * RPT + r * ZR, ZR), :])

    plsc.subcore_barrier()

    def gather(j, buf, sem):
        return pltpu.async_copy(g_hbm.at[src_v.at[pl.ds(j * CH, CH)]], buf, sem)

    def gather_wait(buf, sem):
        pltpu.make_async_copy(g_hbm.at[src_v.at[pl.ds(0, CH)]], buf, sem).wait()

    def scatter(j, buf, sem):
        return pltpu.async_copy(buf, acc.at[dst_v.at[pl.ds(j * CH, CH)]], sem, add=True)

    gather(0, rows_a, sem_a)
    gather(1, rows_b, sem_b)

    def body(g, _):
        j = 2 * g
        gather_wait(rows_a, sem_a)
        dsa = scatter(j, rows_a, sem_sa)
        gather_wait(rows_b, sem_b)
        dsb = scatter(j + 1, rows_b, sem_sb)
        dsa.wait()
        gather(j + 2, rows_a, sem_a)
        dsb.wait()
        gather(j + 3, rows_b, sem_b)
        return 0

    lax.fori_loop(0, (NIT - 1) // 2 - 1, body, 0)
    j_last = NIT - 3
    gather_wait(rows_a, sem_a)
    dsa = scatter(j_last, rows_a, sem_sa)
    gather_wait(rows_b, sem_b)
    dsb = scatter(j_last + 1, rows_b, sem_sb)
    dsa.wait()
    gather(j_last + 2, rows_a, sem_a)
    dsb.wait()
    gather_wait(rows_a, sem_a)
    scatter(NIT - 1, rows_a, sem_sa).wait()
    plsc.subcore_barrier()

    @pl.when(s < NZW)
    def _writeout():
        pltpu.sync_copy(
            acc.at[pl.ds(s * RPT, RPT), :], out_hbm.at[c, pl.ds(s * RPT, RPT), :]
        )


def _make_sc_deg(interpret=False):
    return pl.kernel(
        _sc_deg_body,
        out_type=jax.ShapeDtypeStruct((NC * N,), _f32),
        mesh=_MESH,
        scratch_types=[
            pltpu.VMEM((EPT,), jnp.int32),
            pltpu.VMEM((CH,), _f32),
            pltpu.VMEM((ZBUF1,), _f32),
            pltpu.VMEM_SHARED((N,), _f32),
            pltpu.SemaphoreType.DMA,
        ],
        interpret=interpret,
    )


def _make_sc_segsum(interpret=False):
    return pl.kernel(
        _sc_segsum_body,
        out_type=jax.ShapeDtypeStruct((NC, N, D), _f32),
        mesh=_MESH,
        scratch_types=[
            pltpu.VMEM((EPT,), jnp.int32),
            pltpu.VMEM((EPT,), jnp.int32),
            pltpu.VMEM((CH, D), _f32),
            pltpu.VMEM((CH, D), _f32),
            pltpu.VMEM((ZR, D), _f32),
            pltpu.VMEM_SHARED((N, D), _f32),
            pltpu.SemaphoreType.DMA,
            pltpu.SemaphoreType.DMA,
            pltpu.SemaphoreType.DMA,
            pltpu.SemaphoreType.DMA,
        ],
        interpret=interpret,
    )


_sc_deg = _make_sc_deg()
_sc_segsum = _make_sc_segsum()


_TCB = 1000
_GRID = N // _TCB


def _tc_mm_body(x_ref, w_ref, z_ref):
    z_ref[...] = jnp.dot(x_ref[...], w_ref[...], preferred_element_type=_f32)


def _tc_mm(x, w):
    return pl.pallas_call(
        _tc_mm_body,
        grid=(_GRID,),
        in_specs=[
            pl.BlockSpec((_TCB, D), lambda i: (i, 0)),
            pl.BlockSpec((D, D), lambda i: (0, 0)),
        ],
        out_specs=pl.BlockSpec((_TCB, D), lambda i: (i, 0)),
        out_shape=jax.ShapeDtypeStruct((N, D), _f32),
    )(x, w)


def _tc1_body(g_ref, dp_ref, z_ref, n_ref):
    deg = jnp.maximum(jnp.sum(dp_ref[...], axis=0), 1.0)
    norm = lax.rsqrt(deg)
    z_ref[...] = g_ref[...] * norm
    n_ref[...] = norm


def _tc1(g, dp):
    return pl.pallas_call(
        _tc1_body,
        grid=(_GRID,),
        in_specs=[
            pl.BlockSpec((_TCB, D), lambda i: (i, 0)),
            pl.BlockSpec((NC, _TCB, 1), lambda i: (0, i, 0)),
        ],
        out_specs=[
            pl.BlockSpec((_TCB, D), lambda i: (i, 0)),
            pl.BlockSpec((_TCB, 1), lambda i: (i, 0)),
        ],
        out_shape=[
            jax.ShapeDtypeStruct((N, D), _f32),
            jax.ShapeDtypeStruct((N, 1), _f32),
        ],
    )(g, dp)


def _tc2_body(p0_ref, p1_ref, n_ref, b_ref, w_ref, z_ref):
    norm = n_ref[...]
    h = jnp.maximum((p0_ref[...] + p1_ref[...]) * norm + b_ref[...], 0.0)
    z_ref[...] = jnp.dot(h, w_ref[...], preferred_element_type=_f32) * norm


def _tc2(p0, p1, norm, b1, w2):
    return pl.pallas_call(
        _tc2_body,
        grid=(_GRID,),
        in_specs=[
            pl.BlockSpec((_TCB, D), lambda i: (i, 0)),
            pl.BlockSpec((_TCB, D), lambda i: (i, 0)),
            pl.BlockSpec((_TCB, 1), lambda i: (i, 0)),
            pl.BlockSpec((1, D), lambda i: (0, 0)),
            pl.BlockSpec((D, D), lambda i: (0, 0)),
        ],
        out_specs=pl.BlockSpec((_TCB, D), lambda i: (i, 0)),
        out_shape=jax.ShapeDtypeStruct((N, D), _f32),
    )(p0, p1, norm, b1, w2)


def _tc3_body(p0_ref, p1_ref, n_ref, b_ref, o_ref):
    o_ref[...] = (p0_ref[...] + p1_ref[...]) * n_ref[...] + b_ref[...]


def _tc3(p0, p1, norm, b2):
    return pl.pallas_call(
        _tc3_body,
        grid=(_GRID,),
        in_specs=[
            pl.BlockSpec((_TCB, D), lambda i: (i, 0)),
            pl.BlockSpec((_TCB, D), lambda i: (i, 0)),
            pl.BlockSpec((_TCB, 1), lambda i: (i, 0)),
            pl.BlockSpec((1, D), lambda i: (0, 0)),
        ],
        out_specs=pl.BlockSpec((_TCB, D), lambda i: (i, 0)),
        out_shape=jax.ShapeDtypeStruct((N, D), _f32),
    )(p0, p1, norm, b2)


def kernel(V, E, X, W1, b1, W2, b2):
    src = E[0]
    dst = E[1]
    g1 = _tc_mm(X, W1)
    degp = _sc_deg(dst).reshape(NC, N, 1)
    z1, norm = _tc1(g1, degp)
    p1 = _sc_segsum(z1, src, dst)
    z2 = _tc2(p1[0], p1[1], norm, b1.reshape(1, D), W2)
    p2 = _sc_segsum(z2, src, dst)
    return _tc3(p2[0], p2[1], norm, b2.reshape(1, D))


# speedup vs baseline: 25.1084x; 1.2461x over previous
import functools

import jax
import jax.numpy as jnp
from jax import lax
from jax.experimental import pallas as pl
from jax.experimental.pallas import tpu as pltpu
from jax.experimental.pallas import tpu_sc as plsc

N = 10000
D = 128
NE = 320000

NC = 2
NS = 16
NW = NC * NS

EPT = NE // NW
CH = 80
NIT = EPT // CH
RPT = 1000
NZW = N // RPT
ZR = 40
ZBUF1 = 1008

_MESH = plsc.VectorSubcoreMesh(
    core_axis_name="c", subcore_axis_name="s", num_cores=NC, num_subcores=NS
)

_f32 = jnp.float32


def _sc_deg_body(dst_hbm, out_hbm, idx_v, ones_v, zbuf, acc, sem):
    c = lax.axis_index("c")
    s = lax.axis_index("s")

    def fill_ones(i, _):
        ones_v[pl.ds(i * 16, 16)] = jnp.ones((16,), _f32)
        return 0

    lax.fori_loop(0, CH // 16, fill_ones, 0)

    def fill_zero(i, _):
        zbuf[pl.ds(i * 16, 16)] = jnp.zeros((16,), _f32)
        return 0

    lax.fori_loop(0, ZBUF1 // 16, fill_zero, 0)
    pltpu.sync_copy(dst_hbm.at[pl.ds((c * NS + s) * EPT, EPT)], idx_v)

    @pl.when(s < NZW)
    def _zero():
        pltpu.sync_copy(zbuf.at[pl.ds(0, RPT)], acc.at[pl.ds(s * RPT, RPT)])

    plsc.subcore_barrier()

    def fire(j, _):
        pltpu.async_copy(ones_v, acc.at[idx_v.at[pl.ds(j * CH, CH)]], sem, add=True)
        return 0

    lax.fori_loop(0, NIT, fire, 0)

    def drain(j, _):
        pltpu.make_async_copy(ones_v, acc.at[idx_v.at[pl.ds(0, CH)]], sem).wait()
        return 0

    lax.fori_loop(0, NIT, drain, 0)
    plsc.subcore_barrier()

    @pl.when(s < NZW)
    def _writeout():
        pltpu.sync_copy(acc.at[pl.ds(s * RPT, RPT)], zbuf.at[pl.ds(0, RPT)])
        pltpu.sync_copy(zbuf.at[pl.ds(0, RPT)], out_hbm.at[pl.ds(c * N + s * RPT, RPT)])


SCH = 40
SNIT = EPT // SCH
NBUF = 5
SRNDS = SNIT // NBUF


def _sc_segsum_body(
    g_hbm, src_hbm, dst_hbm, out_hbm,
    src_v, dst_v, b0, b1, b2, b3, b4, acc,
    g0, g1, g2, g3, g4, s0, s1, s2, s3, s4,
):
    bufs = [b0, b1, b2, b3, b4]
    gsems = [g0, g1, g2, g3, g4]
    ssems = [s0, s1, s2, s3, s4]
    c = lax.axis_index("c")
    s = lax.axis_index("s")

    ebase = (c * NS + s) * EPT
    pltpu.sync_copy(src_hbm.at[pl.ds(ebase, EPT)], src_v)
    pltpu.sync_copy(dst_hbm.at[pl.ds(ebase, EPT)], dst_v)

    def fill_zero(i, _):
        def inner(j, _):
            b0[i, pl.ds(j * 16, 16)] = jnp.zeros((16,), _f32)
            return 0

        lax.fori_loop(0, D // 16, inner, 0)
        return 0

    lax.fori_loop(0, SCH, fill_zero, 0)

    @pl.when(s < NZW)
    def _zero():
        for r in range(RPT // SCH):
            pltpu.sync_copy(b0, acc.at[pl.ds(s * RPT + r * SCH, SCH), :])

    plsc.subcore_barrier()

    def gfire(j, b):
        pltpu.async_copy(g_hbm.at[src_v.at[pl.ds(j * SCH, SCH)]], bufs[b], gsems[b])

    def gwait(b):
        pltpu.make_async_copy(
            g_hbm.at[src_v.at[pl.ds(0, SCH)]], bufs[b], gsems[b]
        ).wait()

    def sfire(j, b):
        pltpu.async_copy(
            bufs[b], acc.at[dst_v.at[pl.ds(j * SCH, SCH)]], ssems[b], add=True
        )

    def swait(b):
        pltpu.make_async_copy(
            bufs[b], acc.at[dst_v.at[pl.ds(0, SCH)]], ssems[b]
        ).wait()

    for b in range(NBUF):
        gfire(b, b)

    def body(g, _):
        j0 = g * NBUF
        for b in range(NBUF):
            gwait(b)
            sfire(j0 + b, b)
        for b in range(NBUF):
            swait(b)
            gfire(j0 + NBUF + b, b)
        return 0

    lax.fori_loop(0, SRNDS - 1, body, 0)
    j0 = (SRNDS - 1) * NBUF
    for b in range(NBUF):
        gwait(b)
        sfire(j0 + b, b)
    for b in range(NBUF):
        swait(b)
    plsc.subcore_barrier()

    @pl.when(s < NZW)
    def _writeout():
        pltpu.sync_copy(
            acc.at[pl.ds(s * RPT, RPT), :], out_hbm.at[c, pl.ds(s * RPT, RPT), :]
        )


def _make_sc_deg(interpret=False):
    return pl.kernel(
        _sc_deg_body,
        out_type=jax.ShapeDtypeStruct((NC * N,), _f32),
        mesh=_MESH,
        scratch_types=[
            pltpu.VMEM((EPT,), jnp.int32),
            pltpu.VMEM((CH,), _f32),
            pltpu.VMEM((ZBUF1,), _f32),
            pltpu.VMEM_SHARED((N,), _f32),
            pltpu.SemaphoreType.DMA,
        ],
        interpret=interpret,
    )


def _make_sc_segsum(interpret=False):
    return pl.kernel(
        _sc_segsum_body,
        out_type=jax.ShapeDtypeStruct((NC, N, D), _f32),
        mesh=_MESH,
        scratch_types=(
            [
                pltpu.VMEM((EPT,), jnp.int32),
                pltpu.VMEM((EPT,), jnp.int32),
            ]
            + [pltpu.VMEM((SCH, D), _f32) for _ in range(NBUF)]
            + [pltpu.VMEM_SHARED((N, D), _f32)]
            + [pltpu.SemaphoreType.DMA for _ in range(2 * NBUF)]
        ),
        interpret=interpret,
    )


_sc_deg = _make_sc_deg()
_sc_segsum = _make_sc_segsum()


_TCB = 1000
_GRID = N // _TCB


def _tc_mm_body(x_ref, w_ref, z_ref):
    z_ref[...] = jnp.dot(x_ref[...], w_ref[...], preferred_element_type=_f32)


def _tc_mm(x, w):
    return pl.pallas_call(
        _tc_mm_body,
        grid=(_GRID,),
        in_specs=[
            pl.BlockSpec((_TCB, D), lambda i: (i, 0)),
            pl.BlockSpec((D, D), lambda i: (0, 0)),
        ],
        out_specs=pl.BlockSpec((_TCB, D), lambda i: (i, 0)),
        out_shape=jax.ShapeDtypeStruct((N, D), _f32),
    )(x, w)


def _tc1_body(g_ref, dp_ref, z_ref, n_ref):
    deg = jnp.maximum(jnp.sum(dp_ref[...], axis=0), 1.0)
    norm = lax.rsqrt(deg)
    z_ref[...] = g_ref[...] * norm
    n_ref[...] = norm


def _tc1(g, dp):
    return pl.pallas_call(
        _tc1_body,
        grid=(_GRID,),
        in_specs=[
            pl.BlockSpec((_TCB, D), lambda i: (i, 0)),
            pl.BlockSpec((NC, _TCB, 1), lambda i: (0, i, 0)),
        ],
        out_specs=[
            pl.BlockSpec((_TCB, D), lambda i: (i, 0)),
            pl.BlockSpec((_TCB, 1), lambda i: (i, 0)),
        ],
        out_shape=[
            jax.ShapeDtypeStruct((N, D), _f32),
            jax.ShapeDtypeStruct((N, 1), _f32),
        ],
    )(g, dp)


def _tc2_body(p0_ref, p1_ref, n_ref, b_ref, w_ref, z_ref):
    norm = n_ref[...]
    h = jnp.maximum((p0_ref[...] + p1_ref[...]) * norm + b_ref[...], 0.0)
    z_ref[...] = jnp.dot(h, w_ref[...], preferred_element_type=_f32) * norm


def _tc2(p0, p1, norm, b1, w2):
    return pl.pallas_call(
        _tc2_body,
        grid=(_GRID,),
        in_specs=[
            pl.BlockSpec((_TCB, D), lambda i: (i, 0)),
            pl.BlockSpec((_TCB, D), lambda i: (i, 0)),
            pl.BlockSpec((_TCB, 1), lambda i: (i, 0)),
            pl.BlockSpec((1, D), lambda i: (0, 0)),
            pl.BlockSpec((D, D), lambda i: (0, 0)),
        ],
        out_specs=pl.BlockSpec((_TCB, D), lambda i: (i, 0)),
        out_shape=jax.ShapeDtypeStruct((N, D), _f32),
    )(p0, p1, norm, b1, w2)


def _tc3_body(p0_ref, p1_ref, n_ref, b_ref, o_ref):
    o_ref[...] = (p0_ref[...] + p1_ref[...]) * n_ref[...] + b_ref[...]


def _tc3(p0, p1, norm, b2):
    return pl.pallas_call(
        _tc3_body,
        grid=(_GRID,),
        in_specs=[
            pl.BlockSpec((_TCB, D), lambda i: (i, 0)),
            pl.BlockSpec((_TCB, D), lambda i: (i, 0)),
            pl.BlockSpec((_TCB, 1), lambda i: (i, 0)),
            pl.BlockSpec((1, D), lambda i: (0, 0)),
        ],
        out_specs=pl.BlockSpec((_TCB, D), lambda i: (i, 0)),
        out_shape=jax.ShapeDtypeStruct((N, D), _f32),
    )(p0, p1, norm, b2)


def kernel(V, E, X, W1, b1, W2, b2):
    src = E[0]
    dst = E[1]
    g1 = _tc_mm(X, W1)
    degp = _sc_deg(dst).reshape(NC, N, 1)
    z1, norm = _tc1(g1, degp)
    p1 = _sc_segsum(z1, src, dst)
    z2 = _tc2(p1[0], p1[1], norm, b1.reshape(1, D), W2)
    p2 = _sc_segsum(z2, src, dst)
    return _tc3(p2[0], p2[1], norm, b2.reshape(1, D))


# speedup vs baseline: 26.1558x; 1.0417x over previous
import functools

import jax
import jax.numpy as jnp
from jax import lax
from jax.experimental import pallas as pl
from jax.experimental.pallas import tpu as pltpu
from jax.experimental.pallas import tpu_sc as plsc

N = 10000
D = 128
NE = 320000

NC = 2
NS = 16
NW = NC * NS

EPT = NE // NW
CH = 80
NIT = EPT // CH
RPT = 1000
NZW = N // RPT
ZR = 40
ZBUF1 = 1008

_MESH = plsc.VectorSubcoreMesh(
    core_axis_name="c", subcore_axis_name="s", num_cores=NC, num_subcores=NS
)

_f32 = jnp.float32


def _sc_deg_body(dst_hbm, out_hbm, idx_v, ones_v, zbuf, acc, sem):
    c = lax.axis_index("c")
    s = lax.axis_index("s")

    def fill_ones(i, _):
        ones_v[pl.ds(i * 16, 16)] = jnp.ones((16,), _f32)
        return 0

    lax.fori_loop(0, CH // 16, fill_ones, 0)

    def fill_zero(i, _):
        zbuf[pl.ds(i * 16, 16)] = jnp.zeros((16,), _f32)
        return 0

    lax.fori_loop(0, ZBUF1 // 16, fill_zero, 0)
    pltpu.sync_copy(dst_hbm.at[pl.ds((c * NS + s) * EPT, EPT)], idx_v)

    @pl.when(s < NZW)
    def _zero():
        pltpu.sync_copy(zbuf.at[pl.ds(0, RPT)], acc.at[pl.ds(s * RPT, RPT)])

    plsc.subcore_barrier()

    def fire(j, _):
        pltpu.async_copy(ones_v, acc.at[idx_v.at[pl.ds(j * CH, CH)]], sem, add=True)
        return 0

    lax.fori_loop(0, NIT, fire, 0)

    def drain(j, _):
        pltpu.make_async_copy(ones_v, acc.at[idx_v.at[pl.ds(0, CH)]], sem).wait()
        return 0

    lax.fori_loop(0, NIT, drain, 0)
    plsc.subcore_barrier()

    @pl.when(s < NZW)
    def _writeout():
        pltpu.sync_copy(acc.at[pl.ds(s * RPT, RPT)], zbuf.at[pl.ds(0, RPT)])
        pltpu.sync_copy(zbuf.at[pl.ds(0, RPT)], out_hbm.at[pl.ds(c * N + s * RPT, RPT)])


SCH = 40
SNIT = EPT // SCH
NBUF = 5
SRNDS = SNIT // NBUF


def _sc_segsum_body(
    g_hbm, src_hbm, dst_hbm, out_hbm,
    src_v, dst_v, b0, b1, b2, b3, b4, acc,
    g0, g1, g2, g3, g4, s0, s1, s2, s3, s4,
):
    bufs = [b0, b1, b2, b3, b4]
    gsems = [g0, g1, g2, g3, g4]
    ssems = [s0, s1, s2, s3, s4]
    c = lax.axis_index("c")
    s = lax.axis_index("s")

    ebase = (c * NS + s) * EPT
    d_idx_s = pltpu.async_copy(src_hbm.at[pl.ds(ebase, EPT)], src_v, g0)
    d_idx_d = pltpu.async_copy(dst_hbm.at[pl.ds(ebase, EPT)], dst_v, g1)

    def fill_zero(i, _):
        def inner(j, _):
            b0[i, pl.ds(j * 16, 16)] = jnp.zeros((16,), _f32)
            return 0

        lax.fori_loop(0, D // 16, inner, 0)
        return 0

    lax.fori_loop(0, SCH, fill_zero, 0)

    @pl.when(s < NZW)
    def _zero():
        for r in range(RPT // SCH):
            pltpu.sync_copy(b0, acc.at[pl.ds(s * RPT + r * SCH, SCH), :])

    d_idx_s.wait()
    d_idx_d.wait()
    plsc.subcore_barrier()

    def gfire(j, b):
        pltpu.async_copy(g_hbm.at[src_v.at[pl.ds(j * SCH, SCH)]], bufs[b], gsems[b])

    def gwait(b):
        pltpu.make_async_copy(
            g_hbm.at[src_v.at[pl.ds(0, SCH)]], bufs[b], gsems[b]
        ).wait()

    def sfire(j, b):
        pltpu.async_copy(
            bufs[b], acc.at[dst_v.at[pl.ds(j * SCH, SCH)]], ssems[b], add=True
        )

    def swait(b):
        pltpu.make_async_copy(
            bufs[b], acc.at[dst_v.at[pl.ds(0, SCH)]], ssems[b]
        ).wait()

    for b in range(NBUF):
        gfire(b, b)

    def body(g, _):
        j0 = g * NBUF
        for b in range(NBUF):
            gwait(b)
            sfire(j0 + b, b)
        for b in range(NBUF):
            swait(b)
            gfire(j0 + NBUF + b, b)
        return 0

    lax.fori_loop(0, SRNDS - 1, body, 0)
    j0 = (SRNDS - 1) * NBUF
    for b in range(NBUF):
        gwait(b)
        sfire(j0 + b, b)
    for b in range(NBUF):
        swait(b)
    plsc.subcore_barrier()

    @pl.when(s < NZW)
    def _writeout():
        pltpu.sync_copy(
            acc.at[pl.ds(s * RPT, RPT), :], out_hbm.at[c, pl.ds(s * RPT, RPT), :]
        )


def _make_sc_deg(interpret=False):
    return pl.kernel(
        _sc_deg_body,
        out_type=jax.ShapeDtypeStruct((NC * N,), _f32),
        mesh=_MESH,
        scratch_types=[
            pltpu.VMEM((EPT,), jnp.int32),
            pltpu.VMEM((CH,), _f32),
            pltpu.VMEM((ZBUF1,), _f32),
            pltpu.VMEM_SHARED((N,), _f32),
            pltpu.SemaphoreType.DMA,
        ],
        interpret=interpret,
    )


def _make_sc_segsum(interpret=False):
    return pl.kernel(
        _sc_segsum_body,
        out_type=jax.ShapeDtypeStruct((NC, N, D), _f32),
        mesh=_MESH,
        scratch_types=(
            [
                pltpu.VMEM((EPT,), jnp.int32),
                pltpu.VMEM((EPT,), jnp.int32),
            ]
            + [pltpu.VMEM((SCH, D), _f32) for _ in range(NBUF)]
            + [pltpu.VMEM_SHARED((N, D), _f32)]
            + [pltpu.SemaphoreType.DMA for _ in range(2 * NBUF)]
        ),
        interpret=interpret,
    )


_sc_deg = _make_sc_deg()
_sc_segsum = _make_sc_segsum()


_TCB = 2000
_GRID = N // _TCB


def _tc1_body(x_ref, w_ref, dp_ref, z_ref, n_ref):
    deg = jnp.maximum(jnp.sum(dp_ref[...], axis=0), 1.0)
    norm = lax.rsqrt(deg)
    z = jnp.dot(x_ref[...], w_ref[...], preferred_element_type=_f32)
    z_ref[...] = z * norm
    n_ref[...] = norm


def _tc1(x, w1, dp):
    return pl.pallas_call(
        _tc1_body,
        grid=(_GRID,),
        in_specs=[
            pl.BlockSpec((_TCB, D), lambda i: (i, 0)),
            pl.BlockSpec((D, D), lambda i: (0, 0)),
            pl.BlockSpec((NC, _TCB, 1), lambda i: (0, i, 0)),
        ],
        out_specs=[
            pl.BlockSpec((_TCB, D), lambda i: (i, 0)),
            pl.BlockSpec((_TCB, 1), lambda i: (i, 0)),
        ],
        out_shape=[
            jax.ShapeDtypeStruct((N, D), _f32),
            jax.ShapeDtypeStruct((N, 1), _f32),
        ],
    )(x, w1, dp)


def _tc2_body(p0_ref, p1_ref, n_ref, b_ref, w_ref, z_ref):
    norm = n_ref[...]
    h = jnp.maximum((p0_ref[...] + p1_ref[...]) * norm + b_ref[...], 0.0)
    z_ref[...] = jnp.dot(h, w_ref[...], preferred_element_type=_f32) * norm


def _tc2(p0, p1, norm, b1, w2):
    return pl.pallas_call(
        _tc2_body,
        grid=(_GRID,),
        in_specs=[
            pl.BlockSpec((_TCB, D), lambda i: (i, 0)),
            pl.BlockSpec((_TCB, D), lambda i: (i, 0)),
            pl.BlockSpec((_TCB, 1), lambda i: (i, 0)),
            pl.BlockSpec((1, D), lambda i: (0, 0)),
            pl.BlockSpec((D, D), lambda i: (0, 0)),
        ],
        out_specs=pl.BlockSpec((_TCB, D), lambda i: (i, 0)),
        out_shape=jax.ShapeDtypeStruct((N, D), _f32),
    )(p0, p1, norm, b1, w2)


def _tc3_body(p0_ref, p1_ref, n_ref, b_ref, o_ref):
    o_ref[...] = (p0_ref[...] + p1_ref[...]) * n_ref[...] + b_ref[...]


def _tc3(p0, p1, norm, b2):
    return pl.pallas_call(
        _tc3_body,
        grid=(_GRID,),
        in_specs=[
            pl.BlockSpec((_TCB, D), lambda i: (i, 0)),
            pl.BlockSpec((_TCB, D), lambda i: (i, 0)),
            pl.BlockSpec((_TCB, 1), lambda i: (i, 0)),
            pl.BlockSpec((1, D), lambda i: (0, 0)),
        ],
        out_specs=pl.BlockSpec((_TCB, D), lambda i: (i, 0)),
        out_shape=jax.ShapeDtypeStruct((N, D), _f32),
    )(p0, p1, norm, b2)


def kernel(V, E, X, W1, b1, W2, b2):
    src = E[0]
    dst = E[1]
    degp = _sc_deg(dst).reshape(NC, N, 1)
    z1, norm = _tc1(X, W1, degp)
    p1 = _sc_segsum(z1, src, dst)
    z2 = _tc2(p1[0], p1[1], norm, b1.reshape(1, D), W2)
    p2 = _sc_segsum(z2, src, dst)
    return _tc3(p2[0], p2[1], norm, b2.reshape(1, D))
